# Initial kernel scaffold; baseline (speedup 1.0000x reference)
import jax
import jax.numpy as jnp
from jax.experimental import pallas as pl


def kernel(x, edge_index, edge_attr, W1, as1, ad1, We1, ae1, b1, W2, as2, ad2, We2, ae2, b2):
    raise NotImplementedError("write your pallas kernel here")


# speedup vs baseline: 43.2365x; 43.2365x over previous
import functools

import jax
import jax.numpy as jnp
from jax import lax
from jax.experimental import pallas as pl
from jax.experimental.pallas import tpu as pltpu
from jax.experimental.pallas import tpu_sc as plsc

N = 10000
E = 640000
FIN = 1284
HID = 64

NC, NS, L = 2, 16, 16
NW = NC * NS
ET = E + N
CW = 20352
EP = NW * CW
CWC = 2544
NCH = CW // CWC
B = 128
NEG = -3.0e38

_SC_PARAMS = pltpu.CompilerParams(needs_layout_passes=False,
                                  use_tc_tiling_on_sc=False)
_MESH = plsc.VectorSubcoreMesh(core_axis_name="c", subcore_axis_name="s",
                               num_cores=NC, num_subcores=NS)

BM = 1000


def _tc_proj_body(x_ref, w_ref, a_ref, h_ref, asv_ref, adv_ref):
    h = jnp.dot(x_ref[...], w_ref[...], preferred_element_type=jnp.float32)
    h_ref[...] = h
    asv_ref[...] = jnp.sum(h * a_ref[0][None, :], axis=1, keepdims=True)
    adv_ref[...] = jnp.sum(h * a_ref[1][None, :], axis=1, keepdims=True)


def _tc_proj(x, w1, a_sd):
    return pl.pallas_call(
        _tc_proj_body,
        grid=(N // BM,),
        in_specs=[
            pl.BlockSpec((BM, FIN), lambda i: (i, 0)),
            pl.BlockSpec((FIN, HID), lambda i: (0, 0)),
            pl.BlockSpec((2, HID), lambda i: (0, 0)),
        ],
        out_specs=[
            pl.BlockSpec((BM, HID), lambda i: (i, 0)),
            pl.BlockSpec((BM, 1), lambda i: (i, 0)),
            pl.BlockSpec((BM, 1), lambda i: (i, 0)),
        ],
        out_shape=[
            jax.ShapeDtypeStruct((N, HID), jnp.float32),
            jax.ShapeDtypeStruct((N, 1), jnp.float32),
            jax.ShapeDtypeStruct((N, 1), jnp.float32),
        ],
    )(x, w1, a_sd)


def _make_edge_stats(with_val):
    out_type = []
    if not with_val:
        out_type.append(jax.ShapeDtypeStruct((EP,), jnp.float32))
    out_type.append(jax.ShapeDtypeStruct((NW * L,), jnp.float32))
    out_type.append(jax.ShapeDtypeStruct((NW, N), jnp.float32))
    if with_val:
        out_type.append(jax.ShapeDtypeStruct((NW, N), jnp.float32))

    scratch = [
        pltpu.VMEM((CW,), jnp.int32),
        pltpu.VMEM((CW,), jnp.int32),
        pltpu.VMEM((CW,), jnp.float32),
        pltpu.VMEM((N,), jnp.float32),
        pltpu.VMEM((N,), jnp.float32),
        pltpu.VMEM((N,), jnp.float32),
        pltpu.VMEM((CWC,), jnp.float32),
        pltpu.VMEM((L,), jnp.float32),
    ]
    if with_val:
        scratch.append(pltpu.VMEM((N,), jnp.float32))
        scratch.append(pltpu.VMEM((N,), jnp.float32))

    def body(*refs):
        it = iter(refs)
        sarr, darr, eab, cvec, asv, adv = (next(it) for _ in range(6))
        val = next(it) if with_val else None
        alpha_o = None if with_val else next(it)
        maxes_o, denomp_o = next(it), next(it)
        nump_o = next(it) if with_val else None
        s_t, d_t, alpha_t, asv_t, adv_t, denom_t, eac_b, cv_t = (
            next(it) for _ in range(8))
        val_t = next(it) if with_val else None
        num_t = next(it) if with_val else None

        cid = lax.axis_index("c")
        sid = lax.axis_index("s")
        wid = cid * NS + sid
        base = wid * CW

        pltpu.sync_copy(asv, asv_t)
        pltpu.sync_copy(adv, adv_t)
        if with_val:
            pltpu.sync_copy(val, val_t)
        pltpu.sync_copy(cvec, cv_t)
        pltpu.sync_copy(sarr.at[pl.ds(base, CW)], s_t)
        pltpu.sync_copy(darr.at[pl.ds(base, CW)], d_t)
        cvr = cv_t[...]

        def zbody(i, c):
            denom_t[pl.ds(i * L, L)] = jnp.zeros((L,), jnp.float32)
            if with_val:
                num_t[pl.ds(i * L, L)] = jnp.zeros((L,), jnp.float32)
            return c
        lax.fori_loop(0, N // L, zbody, 0)

        def ch_body(c, mx):
            pltpu.sync_copy(eab.at[pl.ds(base + c * CWC, CWC)], eac_b)

            def vbody(j, mx2):
                off = c * CWC + j * L
                sv = s_t[pl.ds(off, L)]
                dv = d_t[pl.ds(off, L)]
                av = (plsc.load_gather(asv_t, [sv])
                      + plsc.load_gather(adv_t, [dv])
                      + eac_b[pl.ds(j * L, L)] * cvr)
                av = jnp.where(av >= 0.0, av, 0.2 * av)
                gidx = base + off + lax.iota(jnp.int32, 16)
                av = jnp.where(gidx < ET, av, NEG)
                alpha_t[pl.ds(off, L)] = av
                return jnp.maximum(mx2, av)
            return lax.fori_loop(0, CWC // L, vbody, mx)
        mx = lax.fori_loop(0, NCH, ch_body, jnp.full((L,), NEG, jnp.float32))

        eac_b[pl.ds(0, L)] = mx
        pltpu.sync_copy(eac_b.at[pl.ds(0, L)], maxes_o.at[pl.ds(wid * L, L)])

        m = jnp.max(mx)
        msp = jnp.broadcast_to(m, (L,))

        def p2(j, c):
            off = j * L
            av = alpha_t[pl.ds(off, L)]
            ex = jnp.exp(av - msp)
            dv = d_t[pl.ds(off, L)]
            plsc.addupdate_scatter(denom_t, [dv], ex)
            if with_val:
                sv = s_t[pl.ds(off, L)]
                hs = plsc.load_gather(val_t, [sv])
                plsc.addupdate_scatter(num_t, [dv], ex * hs)
            return c
        lax.fori_loop(0, CW // L, p2, 0)

        if not with_val:
            pltpu.sync_copy(alpha_t, alpha_o.at[pl.ds(base, CW)])
        pltpu.sync_copy(denom_t, denomp_o.at[wid])
        if with_val:
            pltpu.sync_copy(num_t, nump_o.at[wid])

    return functools.partial(
        pl.kernel, out_type=tuple(out_type), mesh=_MESH,
        scratch_types=tuple(scratch), compiler_params=_SC_PARAMS)(body)


_edge_stats1 = _make_edge_stats(with_val=False)
_edge_stats2 = _make_edge_stats(with_val=True)


NSL = N // NS
DRC = 125

_AGG_SCRATCH = (
    pltpu.VMEM((NW * L,), jnp.float32),
    pltpu.VMEM((B,), jnp.int32),
    pltpu.VMEM((B,), jnp.int32),
    pltpu.VMEM((B,), jnp.float32),
    pltpu.VMEM((B, HID), jnp.float32),
    pltpu.VMEM((DRC, HID), jnp.float32),
    pltpu.VMEM_SHARED((N, HID), jnp.float32),
    pltpu.SemaphoreType.DMA,
)


def _edge_agg_body(alpha, sarr, darr, maxes, h, nump_o,
                   mx_t, sb, db, ab, rows, zrow, acc, sem):
    cid = lax.axis_index("c")
    sid = lax.axis_index("s")
    wid = cid * NS + sid
    base = wid * CW

    pltpu.sync_copy(maxes, mx_t)

    def mbody(i, mx):
        return jnp.maximum(mx, mx_t[pl.ds(i * L, L)])
    mxv = lax.fori_loop(0, NW, mbody, jnp.full((L,), NEG, jnp.float32))
    g = jnp.max(mxv)
    gsp = jnp.broadcast_to(g, (L,))

    for r in range(DRC):
        for q in range(HID // L):
            zrow[r, pl.ds(q * L, L)] = jnp.zeros((L,), jnp.float32)

    def zc(k, c):
        pltpu.sync_copy(zrow, acc.at[pl.ds(sid * NSL + k * DRC, DRC)])
        return c
    lax.fori_loop(0, NSL // DRC, zc, 0)
    plsc.subcore_barrier()

    def eb(blk, c):
        off = base + blk * B
        pltpu.sync_copy(sarr.at[pl.ds(off, B)], sb)
        pltpu.sync_copy(darr.at[pl.ds(off, B)], db)
        pltpu.sync_copy(alpha.at[pl.ds(off, B)], ab)
        pltpu.async_copy(h.at[sb], rows, sem).wait()
        for j in range(B // L):
            exv = jnp.exp(ab[pl.ds(j * L, L)] - gsp)
            for ee in range(L):
                e = j * L + ee
                esp = jnp.broadcast_to(exv[ee], (L,))
                for q in range(HID // L):
                    rows[e, pl.ds(q * L, L)] = rows[e, pl.ds(q * L, L)] * esp
        pltpu.sync_copy(rows, acc.at[db], add=True)
        return c
    lax.fori_loop(0, CW // B, eb, 0)
    plsc.subcore_barrier()

    def dr(k, c):
        r0 = sid * NSL + k * DRC
        pltpu.sync_copy(acc.at[pl.ds(r0, DRC)], zrow)
        pltpu.sync_copy(zrow, nump_o.at[cid, pl.ds(r0, DRC)])
        return c
    lax.fori_loop(0, NSL // DRC, dr, 0)


_edge_agg = functools.partial(
    pl.kernel,
    out_type=jax.ShapeDtypeStruct((NC, N, HID), jnp.float32),
    mesh=_MESH, scratch_types=_AGG_SCRATCH,
    compiler_params=_SC_PARAMS)(_edge_agg_body)


def _tc_comb1_body(nump_ref, denomp_ref, mx_ref, b1_ref, w2_ref, sc2_ref,
                   h2_ref, asv2_ref, adv2_ref):
    mxw = jnp.max(mx_ref[...], axis=1)
    g = jnp.max(mxw)
    scale = jnp.exp(mxw - g)
    denom = jnp.sum(scale[None, :] * denomp_ref[...], axis=1)
    num = nump_ref[0] + nump_ref[1]
    x1 = num / (denom[:, None] + 1e-16) + b1_ref[0][None, :]
    x2 = jnp.where(x1 > 0.0, x1, jnp.exp(x1) - 1.0)
    h2 = jnp.dot(x2, w2_ref[...], preferred_element_type=jnp.float32)
    h2_ref[...] = h2
    asv2_ref[...] = h2 * sc2_ref[:, 0:1]
    adv2_ref[...] = h2 * sc2_ref[:, 1:2]


def _tc_comb1(nump, denomp, mxs, b1, w2, sc2):
    return pl.pallas_call(
        _tc_comb1_body,
        grid=(N // BM,),
        in_specs=[
            pl.BlockSpec((NC, BM, HID), lambda i: (0, i, 0)),
            pl.BlockSpec((BM, NW), lambda i: (i, 0)),
            pl.BlockSpec((NW, L), lambda i: (0, 0)),
            pl.BlockSpec((1, HID), lambda i: (0, 0)),
            pl.BlockSpec((HID, 1), lambda i: (0, 0)),
            pl.BlockSpec((1, 2), lambda i: (0, 0)),
        ],
        out_specs=[
            pl.BlockSpec((BM, 1), lambda i: (i, 0)),
            pl.BlockSpec((BM, 1), lambda i: (i, 0)),
            pl.BlockSpec((BM, 1), lambda i: (i, 0)),
        ],
        out_shape=[
            jax.ShapeDtypeStruct((N, 1), jnp.float32),
            jax.ShapeDtypeStruct((N, 1), jnp.float32),
            jax.ShapeDtypeStruct((N, 1), jnp.float32),
        ],
    )(nump, denomp, mxs, b1, w2, sc2)


def _tc_comb2_body(nump_ref, denomp_ref, mx_ref, b2_ref, out_ref):
    mxw = jnp.max(mx_ref[...], axis=1)
    g = jnp.max(mxw)
    scale = jnp.exp(mxw - g)
    denom = jnp.sum(scale[None, :] * denomp_ref[...], axis=1)
    num = jnp.sum(scale[None, :] * nump_ref[...], axis=1)
    out = num / (denom + 1e-16) + b2_ref[0, 0]
    out = jnp.where(out > 0.0, out, 0.01 * out)
    out_ref[...] = out[:, None]


def _tc_comb2(nump, denomp, mxs, b2):
    return pl.pallas_call(
        _tc_comb2_body,
        grid=(N // BM,),
        in_specs=[
            pl.BlockSpec((BM, NW), lambda i: (i, 0)),
            pl.BlockSpec((BM, NW), lambda i: (i, 0)),
            pl.BlockSpec((NW, L), lambda i: (0, 0)),
            pl.BlockSpec((1, 1), lambda i: (0, 0)),
        ],
        out_specs=pl.BlockSpec((BM, 1), lambda i: (i, 0)),
        out_shape=jax.ShapeDtypeStruct((N, 1), jnp.float32),
    )(nump, denomp, mxs, b2)


def kernel(x, edge_index, edge_attr, W1, as1, ad1, We1, ae1, b1,
           W2, as2, ad2, We2, ae2, b2):
    src, dst = edge_index[0], edge_index[1]
    loop = jnp.arange(N, dtype=src.dtype)
    ea = edge_attr[:, 0]
    mean_attr = jnp.mean(ea)
    c1 = jnp.sum(We1[0] * ae1)
    c2 = jnp.sum(We2[0] * ae2)

    pad = EP - ET
    sarr = jnp.concatenate([src, loop, jnp.zeros((pad,), src.dtype)])
    darr = jnp.concatenate([dst, loop, jnp.zeros((pad,), dst.dtype)])
    eab = jnp.concatenate([ea, jnp.full((N,), mean_attr, jnp.float32),
                           jnp.zeros((pad,), jnp.float32)])
    cv1 = jnp.broadcast_to(c1, (L,)).astype(jnp.float32)
    cv2 = jnp.broadcast_to(c2, (L,)).astype(jnp.float32)

    h1, asv1, adv1 = _tc_proj(x, W1, jnp.stack([as1, ad1]))
    alpha1, maxes1, denomp1 = _edge_stats1(
        sarr, darr, eab, cv1, asv1[:, 0], adv1[:, 0])
    nump1 = _edge_agg(alpha1, sarr, darr, maxes1, h1)
    h2, asv2, adv2 = _tc_comb1(
        nump1, denomp1.T, maxes1.reshape(NW, L), b1.reshape(1, HID),
        W2, jnp.stack([as2, ad2]).reshape(1, 2))

    maxes2, denomp2, nump2 = _edge_stats2(
        sarr, darr, eab, cv2, asv2[:, 0], adv2[:, 0], h2[:, 0])
    return _tc_comb2(nump2.T, denomp2.T, maxes2.reshape(NW, L),
                     b2.reshape(1, 1))


# speedup vs baseline: 58.2451x; 1.3471x over previous
import functools

import jax
import jax.numpy as jnp
from jax import lax
from jax.experimental import pallas as pl
from jax.experimental.pallas import tpu as pltpu
from jax.experimental.pallas import tpu_sc as plsc

N = 10000
E = 640000
FIN = 1284
HID = 64

NC, NS, L = 2, 16, 16
NW = NC * NS
ET = E + N
CW = 20352
EP = NW * CW
CWC = 2544
NCH = CW // CWC
B = 96
NB = CW // B
NEG = -3.0e38

_SC_PARAMS = pltpu.CompilerParams(needs_layout_passes=False,
                                  use_tc_tiling_on_sc=False)
_MESH = plsc.VectorSubcoreMesh(core_axis_name="c", subcore_axis_name="s",
                               num_cores=NC, num_subcores=NS)

BM = 1000


def _tc_proj_body(x_ref, w_ref, a_ref, h_ref, asv_ref, adv_ref):
    h = jnp.dot(x_ref[...], w_ref[...], preferred_element_type=jnp.float32)
    h_ref[...] = h
    asv_ref[...] = jnp.sum(h * a_ref[0][None, :], axis=1, keepdims=True)
    adv_ref[...] = jnp.sum(h * a_ref[1][None, :], axis=1, keepdims=True)


def _tc_proj(x, w1, a_sd):
    return pl.pallas_call(
        _tc_proj_body,
        grid=(N // BM,),
        in_specs=[
            pl.BlockSpec((BM, FIN), lambda i: (i, 0)),
            pl.BlockSpec((FIN, HID), lambda i: (0, 0)),
            pl.BlockSpec((2, HID), lambda i: (0, 0)),
        ],
        out_specs=[
            pl.BlockSpec((BM, HID), lambda i: (i, 0)),
            pl.BlockSpec((BM, 1), lambda i: (i, 0)),
            pl.BlockSpec((BM, 1), lambda i: (i, 0)),
        ],
        out_shape=[
            jax.ShapeDtypeStruct((N, HID), jnp.float32),
            jax.ShapeDtypeStruct((N, 1), jnp.float32),
            jax.ShapeDtypeStruct((N, 1), jnp.float32),
        ],
    )(x, w1, a_sd)


def _make_edge_stats(with_val):
    out_type = []
    if not with_val:
        out_type.append(jax.ShapeDtypeStruct((EP,), jnp.float32))
    out_type.append(jax.ShapeDtypeStruct((NW * L,), jnp.float32))
    out_type.append(jax.ShapeDtypeStruct((NW, N), jnp.float32))
    if with_val:
        out_type.append(jax.ShapeDtypeStruct((NW, N), jnp.float32))

    scratch = [
        pltpu.VMEM((CW,), jnp.int32),
        pltpu.VMEM((CW,), jnp.int32),
        pltpu.VMEM((CW,), jnp.float32),
        pltpu.VMEM((N,), jnp.float32),
        pltpu.VMEM((N,), jnp.float32),
        pltpu.VMEM((N,), jnp.float32),
        pltpu.VMEM((CWC,), jnp.float32),
        pltpu.VMEM((L,), jnp.float32),
    ]
    if with_val:
        scratch.append(pltpu.VMEM((N,), jnp.float32))
        scratch.append(pltpu.VMEM((N,), jnp.float32))

    def body(*refs):
        it = iter(refs)
        sarr, darr, eab, cvec, asv, adv = (next(it) for _ in range(6))
        val = next(it) if with_val else None
        alpha_o = None if with_val else next(it)
        maxes_o, denomp_o = next(it), next(it)
        nump_o = next(it) if with_val else None
        s_t, d_t, alpha_t, asv_t, adv_t, denom_t, eac_b, cv_t = (
            next(it) for _ in range(8))
        val_t = next(it) if with_val else None
        num_t = next(it) if with_val else None

        cid = lax.axis_index("c")
        sid = lax.axis_index("s")
        wid = cid * NS + sid
        base = wid * CW

        pltpu.sync_copy(asv, asv_t)
        pltpu.sync_copy(adv, adv_t)
        if with_val:
            pltpu.sync_copy(val, val_t)
        pltpu.sync_copy(cvec, cv_t)
        pltpu.sync_copy(sarr.at[pl.ds(base, CW)], s_t)
        pltpu.sync_copy(darr.at[pl.ds(base, CW)], d_t)
        cvr = cv_t[...]

        def zbody(i, c):
            denom_t[pl.ds(i * L, L)] = jnp.zeros((L,), jnp.float32)
            if with_val:
                num_t[pl.ds(i * L, L)] = jnp.zeros((L,), jnp.float32)
            return c
        lax.fori_loop(0, N // L, zbody, 0)

        def ch_body(c, mx):
            pltpu.sync_copy(eab.at[pl.ds(base + c * CWC, CWC)], eac_b)

            def vbody(j, mx2):
                off = c * CWC + j * L
                sv = s_t[pl.ds(off, L)]
                dv = d_t[pl.ds(off, L)]
                av = (plsc.load_gather(asv_t, [sv])
                      + plsc.load_gather(adv_t, [dv])
                      + eac_b[pl.ds(j * L, L)] * cvr)
                av = jnp.where(av >= 0.0, av, 0.2 * av)
                gidx = base + off + lax.iota(jnp.int32, 16)
                av = jnp.where(gidx < ET, av, NEG)
                alpha_t[pl.ds(off, L)] = av
                return jnp.maximum(mx2, av)
            return lax.fori_loop(0, CWC // L, vbody, mx)
        mx = lax.fori_loop(0, NCH, ch_body, jnp.full((L,), NEG, jnp.float32))

        eac_b[pl.ds(0, L)] = mx
        pltpu.sync_copy(eac_b.at[pl.ds(0, L)], maxes_o.at[pl.ds(wid * L, L)])

        m = jnp.max(mx)
        msp = jnp.broadcast_to(m, (L,))

        def p2(j, c):
            off = j * L
            av = alpha_t[pl.ds(off, L)]
            ex = jnp.exp(av - msp)
            dv = d_t[pl.ds(off, L)]
            plsc.addupdate_scatter(denom_t, [dv], ex)
            if with_val:
                sv = s_t[pl.ds(off, L)]
                hs = plsc.load_gather(val_t, [sv])
                plsc.addupdate_scatter(num_t, [dv], ex * hs)
            return c
        lax.fori_loop(0, CW // L, p2, 0)

        if not with_val:
            pltpu.sync_copy(alpha_t, alpha_o.at[pl.ds(base, CW)])
        pltpu.sync_copy(denom_t, denomp_o.at[wid])
        if with_val:
            pltpu.sync_copy(num_t, nump_o.at[wid])

    return functools.partial(
        pl.kernel, out_type=tuple(out_type), mesh=_MESH,
        scratch_types=tuple(scratch), compiler_params=_SC_PARAMS)(body)


_edge_stats1 = _make_edge_stats(with_val=False)
_edge_stats2 = _make_edge_stats(with_val=True)


NSL = N // NS
DRC = 125

_AGG_SCRATCH = (
    pltpu.VMEM((NW * L,), jnp.float32),
    pltpu.VMEM((2, B), jnp.int32),
    pltpu.VMEM((2, B), jnp.int32),
    pltpu.VMEM((2, B), jnp.float32),
    pltpu.VMEM((B, HID), jnp.float32),
    pltpu.VMEM((B, HID), jnp.float32),
    pltpu.VMEM((DRC, HID), jnp.float32),
    pltpu.VMEM_SHARED((N, HID), jnp.float32),
    pltpu.SemaphoreType.DMA,
    pltpu.SemaphoreType.DMA,
    pltpu.SemaphoreType.DMA,
    pltpu.SemaphoreType.DMA,
    pltpu.SemaphoreType.DMA,
    pltpu.SemaphoreType.DMA,
)


def _edge_agg_body(alpha, sarr, darr, maxes, h, nump_o,
                   mx_t, sb, db, ab, rows0, rows1, zrow, acc,
                   gs0, gs1, ss0, ss1, is0, is1):
    cid = lax.axis_index("c")
    sid = lax.axis_index("s")
    wid = cid * NS + sid
    base = wid * CW

    pltpu.sync_copy(maxes, mx_t)

    def mbody(i, mx):
        return jnp.maximum(mx, mx_t[pl.ds(i * L, L)])
    mxv = lax.fori_loop(0, NW, mbody, jnp.full((L,), NEG, jnp.float32))
    g = jnp.max(mxv)
    gsp = jnp.broadcast_to(g, (L,))

    for r in range(DRC):
        for q in range(HID // L):
            zrow[r, pl.ds(q * L, L)] = jnp.zeros((L,), jnp.float32)

    def zc(k, c):
        pltpu.sync_copy(zrow, acc.at[pl.ds(sid * NSL + k * DRC, DRC)])
        return c
    lax.fori_loop(0, NSL // DRC, zc, 0)
    plsc.subcore_barrier()

    rows = (rows0, rows1)
    gsems = (gs0, gs1)
    ssems = (ss0, ss1)
    isems = (is0, is1)

    def idx_start(p, blk):
        off = base + blk * B
        pltpu.async_copy(sarr.at[pl.ds(off, B)], sb.at[p], isems[p])
        pltpu.async_copy(darr.at[pl.ds(off, B)], db.at[p], isems[p])
        pltpu.async_copy(alpha.at[pl.ds(off, B)], ab.at[p], isems[p])

    def idx_wait(p):
        pltpu.make_async_copy(sarr.at[pl.ds(base, B)], sb.at[p], isems[p]).wait()
        pltpu.make_async_copy(darr.at[pl.ds(base, B)], db.at[p], isems[p]).wait()
        pltpu.make_async_copy(alpha.at[pl.ds(base, B)], ab.at[p], isems[p]).wait()

    def gather_start(p):
        pltpu.async_copy(h.at[sb.at[p]], rows[p], gsems[p])

    def gather_wait(p):
        pltpu.make_async_copy(h.at[sb.at[p]], rows[p], gsems[p]).wait()

    def scatter_start(p):
        pltpu.async_copy(rows[p], acc.at[db.at[p]], ssems[p], add=True)

    def scatter_wait(p):
        pltpu.make_async_copy(rows[p], acc.at[db.at[p]], ssems[p]).wait()

    def process(p):
        for j in range(B // L):
            exv = jnp.exp(ab[p, pl.ds(j * L, L)] - gsp)
            for ee in range(L):
                e = j * L + ee
                esp = jnp.broadcast_to(exv[ee], (L,))
                for q in range(HID // L):
                    rows[p][e, pl.ds(q * L, L)] = (
                        rows[p][e, pl.ds(q * L, L)] * esp)

    idx_start(0, jnp.int32(0))
    idx_wait(0)
    gather_start(0)
    idx_start(1, jnp.int32(1))
    idx_wait(1)
    gather_wait(0)
    gather_start(1)
    process(0)
    scatter_start(0)
    idx_start(0, jnp.int32(2))
    gather_wait(1)
    process(1)
    scatter_start(1)
    idx_start(1, jnp.int32(3))

    def pair_body(gg, carry):
        b0 = gg * 2
        scatter_wait(0)
        idx_wait(0)
        gather_start(0)
        scatter_wait(1)
        idx_wait(1)
        gather_start(1)
        gather_wait(0)
        process(0)
        scatter_start(0)

        @pl.when(b0 + 2 < NB)
        def _():
            idx_start(0, b0 + 2)
        gather_wait(1)
        process(1)
        scatter_start(1)

        @pl.when(b0 + 3 < NB)
        def _():
            idx_start(1, b0 + 3)
        return carry

    lax.fori_loop(1, NB // 2, pair_body, 0)
    scatter_wait(0)
    scatter_wait(1)
    plsc.subcore_barrier()

    def dr(k, c):
        r0 = sid * NSL + k * DRC
        pltpu.sync_copy(acc.at[pl.ds(r0, DRC)], zrow)
        pltpu.sync_copy(zrow, nump_o.at[cid, pl.ds(r0, DRC)])
        return c
    lax.fori_loop(0, NSL // DRC, dr, 0)


_edge_agg = functools.partial(
    pl.kernel,
    out_type=jax.ShapeDtypeStruct((NC, N, HID), jnp.float32),
    mesh=_MESH, scratch_types=_AGG_SCRATCH,
    compiler_params=_SC_PARAMS)(_edge_agg_body)


def _tc_comb1_body(nump_ref, denomp_ref, mx_ref, b1_ref, w2_ref, sc2_ref,
                   h2_ref, asv2_ref, adv2_ref):
    mxw = jnp.max(mx_ref[...], axis=1)
    g = jnp.max(mxw)
    scale = jnp.exp(mxw - g)
    denom = jnp.sum(scale[None, :] * denomp_ref[...], axis=1)
    num = nump_ref[0] + nump_ref[1]
    x1 = num / (denom[:, None] + 1e-16) + b1_ref[0][None, :]
    x2 = jnp.where(x1 > 0.0, x1, jnp.exp(x1) - 1.0)
    h2 = jnp.dot(x2, w2_ref[...], preferred_element_type=jnp.float32)
    h2_ref[...] = h2
    asv2_ref[...] = h2 * sc2_ref[:, 0:1]
    adv2_ref[...] = h2 * sc2_ref[:, 1:2]


def _tc_comb1(nump, denomp, mxs, b1, w2, sc2):
    return pl.pallas_call(
        _tc_comb1_body,
        grid=(N // BM,),
        in_specs=[
            pl.BlockSpec((NC, BM, HID), lambda i: (0, i, 0)),
            pl.BlockSpec((BM, NW), lambda i: (i, 0)),
            pl.BlockSpec((NW, L), lambda i: (0, 0)),
            pl.BlockSpec((1, HID), lambda i: (0, 0)),
            pl.BlockSpec((HID, 1), lambda i: (0, 0)),
            pl.BlockSpec((1, 2), lambda i: (0, 0)),
        ],
        out_specs=[
            pl.BlockSpec((BM, 1), lambda i: (i, 0)),
            pl.BlockSpec((BM, 1), lambda i: (i, 0)),
            pl.BlockSpec((BM, 1), lambda i: (i, 0)),
        ],
        out_shape=[
            jax.ShapeDtypeStruct((N, 1), jnp.float32),
            jax.ShapeDtypeStruct((N, 1), jnp.float32),
            jax.ShapeDtypeStruct((N, 1), jnp.float32),
        ],
    )(nump, denomp, mxs, b1, w2, sc2)


def _tc_comb2_body(nump_ref, denomp_ref, mx_ref, b2_ref, out_ref):
    mxw = jnp.max(mx_ref[...], axis=1)
    g = jnp.max(mxw)
    scale = jnp.exp(mxw - g)
    denom = jnp.sum(scale[None, :] * denomp_ref[...], axis=1)
    num = jnp.sum(scale[None, :] * nump_ref[...], axis=1)
    out = num / (denom + 1e-16) + b2_ref[0, 0]
    out = jnp.where(out > 0.0, out, 0.01 * out)
    out_ref[...] = out[:, None]


def _tc_comb2(nump, denomp, mxs, b2):
    return pl.pallas_call(
        _tc_comb2_body,
        grid=(N // BM,),
        in_specs=[
            pl.BlockSpec((BM, NW), lambda i: (i, 0)),
            pl.BlockSpec((BM, NW), lambda i: (i, 0)),
            pl.BlockSpec((NW, L), lambda i: (0, 0)),
            pl.BlockSpec((1, 1), lambda i: (0, 0)),
        ],
        out_specs=pl.BlockSpec((BM, 1), lambda i: (i, 0)),
        out_shape=jax.ShapeDtypeStruct((N, 1), jnp.float32),
    )(nump, denomp, mxs, b2)


def kernel(x, edge_index, edge_attr, W1, as1, ad1, We1, ae1, b1,
           W2, as2, ad2, We2, ae2, b2):
    src, dst = edge_index[0], edge_index[1]
    loop = jnp.arange(N, dtype=src.dtype)
    ea = edge_attr[:, 0]
    mean_attr = jnp.mean(ea)
    c1 = jnp.sum(We1[0] * ae1)
    c2 = jnp.sum(We2[0] * ae2)

    pad = EP - ET
    sarr = jnp.concatenate([src, loop, jnp.zeros((pad,), src.dtype)])
    darr = jnp.concatenate([dst, loop, jnp.zeros((pad,), dst.dtype)])
    eab = jnp.concatenate([ea, jnp.full((N,), mean_attr, jnp.float32),
                           jnp.zeros((pad,), jnp.float32)])
    cv1 = jnp.broadcast_to(c1, (L,)).astype(jnp.float32)
    cv2 = jnp.broadcast_to(c2, (L,)).astype(jnp.float32)

    h1, asv1, adv1 = _tc_proj(x, W1, jnp.stack([as1, ad1]))
    alpha1, maxes1, denomp1 = _edge_stats1(
        sarr, darr, eab, cv1, asv1[:, 0], adv1[:, 0])
    nump1 = _edge_agg(alpha1, sarr, darr, maxes1, h1)
    h2, asv2, adv2 = _tc_comb1(
        nump1, denomp1.T, maxes1.reshape(NW, L), b1.reshape(1, HID),
        W2, jnp.stack([as2, ad2]).reshape(1, 2))

    maxes2, denomp2, nump2 = _edge_stats2(
        sarr, darr, eab, cv2, asv2[:, 0], adv2[:, 0], h2[:, 0])
    return _tc_comb2(nump2.T, denomp2.T, maxes2.reshape(NW, L),
                     b2.reshape(1, 1))


# speedup vs baseline: 64.4515x; 1.1066x over previous
import functools

import jax
import jax.numpy as jnp
from jax import lax
from jax.experimental import pallas as pl
from jax.experimental.pallas import tpu as pltpu
from jax.experimental.pallas import tpu_sc as plsc

N = 10000
E = 640000
FIN = 1284
HID = 64

NC, NS, L = 2, 16, 16
NW = NC * NS
ET = E + N
CW = 20352
EP = NW * CW
CWC = 2544
NCH = CW // CWC
B = 96
NB = CW // B
NEG = -3.0e38

_SC_PARAMS = pltpu.CompilerParams(needs_layout_passes=False,
                                  use_tc_tiling_on_sc=False)
_MESH = plsc.VectorSubcoreMesh(core_axis_name="c", subcore_axis_name="s",
                               num_cores=NC, num_subcores=NS)

BM = 1000


def _tc_proj_body(x_ref, w_ref, a_ref, h_ref, asv_ref, adv_ref):
    h = jnp.dot(x_ref[...], w_ref[...], preferred_element_type=jnp.float32)
    h_ref[...] = h
    asv_ref[...] = jnp.sum(h * a_ref[0][None, :], axis=1, keepdims=True)
    adv_ref[...] = jnp.sum(h * a_ref[1][None, :], axis=1, keepdims=True)


def _tc_proj(x, w1, a_sd):
    return pl.pallas_call(
        _tc_proj_body,
        grid=(N // BM,),
        in_specs=[
            pl.BlockSpec((BM, FIN), lambda i: (i, 0)),
            pl.BlockSpec((FIN, HID), lambda i: (0, 0)),
            pl.BlockSpec((2, HID), lambda i: (0, 0)),
        ],
        out_specs=[
            pl.BlockSpec((BM, HID), lambda i: (i, 0)),
            pl.BlockSpec((BM, 1), lambda i: (i, 0)),
            pl.BlockSpec((BM, 1), lambda i: (i, 0)),
        ],
        out_shape=[
            jax.ShapeDtypeStruct((N, HID), jnp.float32),
            jax.ShapeDtypeStruct((N, 1), jnp.float32),
            jax.ShapeDtypeStruct((N, 1), jnp.float32),
        ],
    )(x, w1, a_sd)


def _make_edge_stats(with_val):
    out_type = []
    if not with_val:
        out_type.append(jax.ShapeDtypeStruct((EP,), jnp.float32))
    out_type.append(jax.ShapeDtypeStruct((NW * L,), jnp.float32))
    out_type.append(jax.ShapeDtypeStruct((NW, N), jnp.float32))
    if with_val:
        out_type.append(jax.ShapeDtypeStruct((NW, N), jnp.float32))

    scratch = [
        pltpu.VMEM((CW,), jnp.int32),
        pltpu.VMEM((CW,), jnp.int32),
        pltpu.VMEM((CW,), jnp.float32),
        pltpu.VMEM((N,), jnp.float32),
        pltpu.VMEM((N,), jnp.float32),
        pltpu.VMEM((N,), jnp.float32),
        pltpu.VMEM((CWC,), jnp.float32),
        pltpu.VMEM((L,), jnp.float32),
    ]
    if with_val:
        scratch.append(pltpu.VMEM((N,), jnp.float32))
        scratch.append(pltpu.VMEM((N,), jnp.float32))

    def body(*refs):
        it = iter(refs)
        sarr, darr, eab, cvec, asv, adv = (next(it) for _ in range(6))
        val = next(it) if with_val else None
        alpha_o = None if with_val else next(it)
        maxes_o, denomp_o = next(it), next(it)
        nump_o = next(it) if with_val else None
        s_t, d_t, alpha_t, asv_t, adv_t, denom_t, eac_b, cv_t = (
            next(it) for _ in range(8))
        val_t = next(it) if with_val else None
        num_t = next(it) if with_val else None

        cid = lax.axis_index("c")
        sid = lax.axis_index("s")
        wid = cid * NS + sid
        base = wid * CW

        pltpu.sync_copy(asv, asv_t)
        pltpu.sync_copy(adv, adv_t)
        if with_val:
            pltpu.sync_copy(val, val_t)
        pltpu.sync_copy(cvec, cv_t)
        pltpu.sync_copy(sarr.at[pl.ds(base, CW)], s_t)
        pltpu.sync_copy(darr.at[pl.ds(base, CW)], d_t)
        cvr = cv_t[...]

        def zbody(i, c):
            denom_t[pl.ds(i * L, L)] = jnp.zeros((L,), jnp.float32)
            if with_val:
                num_t[pl.ds(i * L, L)] = jnp.zeros((L,), jnp.float32)
            return c
        lax.fori_loop(0, N // L, zbody, 0)

        def ch_body(c, mx):
            pltpu.sync_copy(eab.at[pl.ds(base + c * CWC, CWC)], eac_b)

            def vbody(j, mx2):
                off = c * CWC + j * L
                sv = s_t[pl.ds(off, L)]
                dv = d_t[pl.ds(off, L)]
                av = (plsc.load_gather(asv_t, [sv])
                      + plsc.load_gather(adv_t, [dv])
                      + eac_b[pl.ds(j * L, L)] * cvr)
                av = jnp.where(av >= 0.0, av, 0.2 * av)
                gidx = base + off + lax.iota(jnp.int32, 16)
                av = jnp.where(gidx < ET, av, NEG)
                alpha_t[pl.ds(off, L)] = av
                return jnp.maximum(mx2, av)
            return lax.fori_loop(0, CWC // L, vbody, mx)
        mx = lax.fori_loop(0, NCH, ch_body, jnp.full((L,), NEG, jnp.float32))

        eac_b[pl.ds(0, L)] = mx
        pltpu.sync_copy(eac_b.at[pl.ds(0, L)], maxes_o.at[pl.ds(wid * L, L)])

        m = jnp.max(mx)
        msp = jnp.broadcast_to(m, (L,))

        def p2(j, c):
            off = j * L
            av = alpha_t[pl.ds(off, L)]
            ex = jnp.exp(av - msp)
            dv = d_t[pl.ds(off, L)]
            plsc.addupdate_scatter(denom_t, [dv], ex)
            if with_val:
                sv = s_t[pl.ds(off, L)]
                hs = plsc.load_gather(val_t, [sv])
                plsc.addupdate_scatter(num_t, [dv], ex * hs)
            return c
        lax.fori_loop(0, CW // L, p2, 0)

        if not with_val:
            pltpu.sync_copy(alpha_t, alpha_o.at[pl.ds(base, CW)])
        pltpu.sync_copy(denom_t, denomp_o.at[wid])
        if with_val:
            pltpu.sync_copy(num_t, nump_o.at[wid])

    return functools.partial(
        pl.kernel, out_type=tuple(out_type), mesh=_MESH,
        scratch_types=tuple(scratch), compiler_params=_SC_PARAMS)(body)


_edge_stats2 = _make_edge_stats(with_val=True)


NSL = N // NS
DRC = 25

_FUSED_SCRATCH = (
    pltpu.VMEM((CW,), jnp.int32),
    pltpu.VMEM((2, B), jnp.int32),
    pltpu.VMEM((CW,), jnp.float32),
    pltpu.VMEM((N,), jnp.float32),
    pltpu.VMEM((N,), jnp.float32),
    pltpu.VMEM((N,), jnp.float32),
    pltpu.VMEM((CWC,), jnp.float32),
    pltpu.VMEM((L,), jnp.float32),
    pltpu.VMEM((NS * L,), jnp.float32),
    pltpu.VMEM((B, HID), jnp.float32),
    pltpu.VMEM((B, HID), jnp.float32),
    pltpu.VMEM((DRC, HID), jnp.float32),
    pltpu.VMEM_SHARED((N, HID), jnp.float32),
    pltpu.VMEM_SHARED((NS * L,), jnp.float32),
    pltpu.SemaphoreType.DMA,
    pltpu.SemaphoreType.DMA,
    pltpu.SemaphoreType.DMA,
    pltpu.SemaphoreType.DMA,
)

_MASK16 = jnp.int32(0xFFFF)


def _edge_fused1_body(sdarr, eab, cvec, asv, adv, h,
                      maxes_o, denomp_o, nump_o,
                      sd_t, sidx, alpha_t, asv_t, adv_t, denom_t, eac_b,
                      cv_t, mxv_t, rows0, rows1, zrow, acc, mxsh,
                      gs0, gs1, ss0, ss1):
    cid = lax.axis_index("c")
    sid = lax.axis_index("s")
    wid = cid * NS + sid
    base = wid * CW

    pltpu.sync_copy(asv, asv_t)
    pltpu.sync_copy(adv, adv_t)
    pltpu.sync_copy(cvec, cv_t)
    pltpu.sync_copy(sdarr.at[pl.ds(base, CW)], sd_t)
    cvr = cv_t[...]

    def zb(i, c):
        denom_t[pl.ds(i * L, L)] = jnp.zeros((L,), jnp.float32)
        return c
    lax.fori_loop(0, N // L, zb, 0)

    def ch_body(c, mx):
        pltpu.sync_copy(eab.at[pl.ds(base + c * CWC, CWC)], eac_b)

        def vbody(j, mx2):
            off = c * CWC + j * L
            pk = sd_t[pl.ds(off, L)]
            sv = lax.shift_right_logical(pk, 16)
            dv = pk & _MASK16
            av = (plsc.load_gather(asv_t, [sv])
                  + plsc.load_gather(adv_t, [dv])
                  + eac_b[pl.ds(j * L, L)] * cvr)
            av = jnp.where(av >= 0.0, av, 0.2 * av)
            gidx = base + off + lax.iota(jnp.int32, 16)
            av = jnp.where(gidx < ET, av, NEG)
            alpha_t[pl.ds(off, L)] = av
            return jnp.maximum(mx2, av)
        return lax.fori_loop(0, CWC // L, vbody, mx)
    mx = lax.fori_loop(0, NCH, ch_body, jnp.full((L,), NEG, jnp.float32))

    eac_b[pl.ds(0, L)] = mx
    pltpu.sync_copy(eac_b.at[pl.ds(0, L)], maxes_o.at[pl.ds(wid * L, L)])
    pltpu.sync_copy(eac_b.at[pl.ds(0, L)], mxsh.at[pl.ds(sid * L, L)])

    for r in range(DRC):
        for q in range(HID // L):
            zrow[r, pl.ds(q * L, L)] = jnp.zeros((L,), jnp.float32)

    def zc(k, c):
        pltpu.sync_copy(zrow, acc.at[pl.ds(sid * NSL + k * DRC, DRC)])
        return c
    lax.fori_loop(0, NSL // DRC, zc, 0)
    plsc.subcore_barrier()

    pltpu.sync_copy(mxsh, mxv_t)

    def mb(i, m):
        return jnp.maximum(m, mxv_t[pl.ds(i * L, L)])
    mcv = lax.fori_loop(0, NS, mb, jnp.full((L,), NEG, jnp.float32))
    msp = jnp.broadcast_to(jnp.max(mcv), (L,))

    rows = (rows0, rows1)
    gsems = (gs0, gs1)
    ssems = (ss0, ss1)

    def gather_start(p, blk):
        for q in range(B // L):
            pk = sd_t[pl.ds(blk * B + q * L, L)]
            sidx[p, pl.ds(q * L, L)] = lax.shift_right_logical(pk, 16)
        pltpu.async_copy(h.at[sidx.at[p]], rows[p], gsems[p])

    def gather_wait(p):
        pltpu.make_async_copy(h.at[sidx.at[p]], rows[p], gsems[p]).wait()

    def process(p, blk):
        for j in range(B // L):
            off = blk * B + j * L
            exv = jnp.exp(alpha_t[pl.ds(off, L)] - msp)
            dv = sd_t[pl.ds(off, L)] & _MASK16
            plsc.addupdate_scatter(denom_t, [dv], exv)
            for ee in range(L):
                e = j * L + ee
                esp = jnp.broadcast_to(exv[ee], (L,))
                for q in range(HID // L):
                    rows[p][e, pl.ds(q * L, L)] = (
                        rows[p][e, pl.ds(q * L, L)] * esp)
            pltpu.async_copy(rows[p].at[pl.ds(j * L, L)], acc.at[dv],
                             ssems[p], add=True)

    def scatter_wait(p):
        pltpu.make_async_copy(rows[p], acc.at[sidx.at[p]], ssems[p]).wait()

    gather_start(0, jnp.int32(0))
    gather_start(1, jnp.int32(1))

    def pair_body(gg, carry):
        b0 = gg * 2
        gather_wait(0)
        process(0, b0)
        gather_wait(1)
        process(1, b0 + 1)
        scatter_wait(0)

        @pl.when(b0 + 2 < NB)
        def _():
            gather_start(0, b0 + 2)
        scatter_wait(1)

        @pl.when(b0 + 3 < NB)
        def _():
            gather_start(1, b0 + 3)
        return carry

    lax.fori_loop(0, NB // 2, pair_body, 0)
    plsc.subcore_barrier()

    pltpu.sync_copy(denom_t, denomp_o.at[wid])

    def dr(k, c):
        r0 = sid * NSL + k * DRC
        pltpu.sync_copy(acc.at[pl.ds(r0, DRC)], zrow)
        pltpu.sync_copy(zrow, nump_o.at[cid, pl.ds(r0, DRC)])
        return c
    lax.fori_loop(0, NSL // DRC, dr, 0)


_edge_fused1 = functools.partial(
    pl.kernel,
    out_type=(jax.ShapeDtypeStruct((NW * L,), jnp.float32),
              jax.ShapeDtypeStruct((NW, N), jnp.float32),
              jax.ShapeDtypeStruct((NC, N, HID), jnp.float32)),
    mesh=_MESH, scratch_types=_FUSED_SCRATCH,
    compiler_params=_SC_PARAMS)(_edge_fused1_body)


def _tc_comb1_body(nump_ref, denomp_ref, mx_ref, b1_ref, w2_ref, sc2_ref,
                   h2_ref, asv2_ref, adv2_ref):
    mc0 = jnp.max(mx_ref[0:NS, :])
    mc1 = jnp.max(mx_ref[NS:NW, :])
    g = jnp.maximum(mc0, mc1)
    e0 = jnp.exp(mc0 - g)
    e1 = jnp.exp(mc1 - g)
    dp = denomp_ref[...]
    denom = (e0 * jnp.sum(dp[:, 0:NS], axis=1)
             + e1 * jnp.sum(dp[:, NS:NW], axis=1))
    num = e0 * nump_ref[0] + e1 * nump_ref[1]
    x1 = num / (denom[:, None] + 1e-16) + b1_ref[0][None, :]
    x2 = jnp.where(x1 > 0.0, x1, jnp.exp(x1) - 1.0)
    h2 = jnp.dot(x2, w2_ref[...], preferred_element_type=jnp.float32)
    h2_ref[...] = h2
    asv2_ref[...] = h2 * sc2_ref[:, 0:1]
    adv2_ref[...] = h2 * sc2_ref[:, 1:2]


def _tc_comb1(nump, denomp, mxs, b1, w2, sc2):
    return pl.pallas_call(
        _tc_comb1_body,
        grid=(N // BM,),
        in_specs=[
            pl.BlockSpec((NC, BM, HID), lambda i: (0, i, 0)),
            pl.BlockSpec((BM, NW), lambda i: (i, 0)),
            pl.BlockSpec((NW, L), lambda i: (0, 0)),
            pl.BlockSpec((1, HID), lambda i: (0, 0)),
            pl.BlockSpec((HID, 1), lambda i: (0, 0)),
            pl.BlockSpec((1, 2), lambda i: (0, 0)),
        ],
        out_specs=[
            pl.BlockSpec((BM, 1), lambda i: (i, 0)),
            pl.BlockSpec((BM, 1), lambda i: (i, 0)),
            pl.BlockSpec((BM, 1), lambda i: (i, 0)),
        ],
        out_shape=[
            jax.ShapeDtypeStruct((N, 1), jnp.float32),
            jax.ShapeDtypeStruct((N, 1), jnp.float32),
            jax.ShapeDtypeStruct((N, 1), jnp.float32),
        ],
    )(nump, denomp, mxs, b1, w2, sc2)


def _tc_comb2_body(nump_ref, denomp_ref, mx_ref, b2_ref, out_ref):
    mxw = jnp.max(mx_ref[...], axis=1)
    g = jnp.max(mxw)
    scale = jnp.exp(mxw - g)
    denom = jnp.sum(scale[None, :] * denomp_ref[...], axis=1)
    num = jnp.sum(scale[None, :] * nump_ref[...], axis=1)
    out = num / (denom + 1e-16) + b2_ref[0, 0]
    out = jnp.where(out > 0.0, out, 0.01 * out)
    out_ref[...] = out[:, None]


def _tc_comb2(nump, denomp, mxs, b2):
    return pl.pallas_call(
        _tc_comb2_body,
        grid=(N // BM,),
        in_specs=[
            pl.BlockSpec((BM, NW), lambda i: (i, 0)),
            pl.BlockSpec((BM, NW), lambda i: (i, 0)),
            pl.BlockSpec((NW, L), lambda i: (0, 0)),
            pl.BlockSpec((1, 1), lambda i: (0, 0)),
        ],
        out_specs=pl.BlockSpec((BM, 1), lambda i: (i, 0)),
        out_shape=jax.ShapeDtypeStruct((N, 1), jnp.float32),
    )(nump, denomp, mxs, b2)


def kernel(x, edge_index, edge_attr, W1, as1, ad1, We1, ae1, b1,
           W2, as2, ad2, We2, ae2, b2):
    src, dst = edge_index[0], edge_index[1]
    loop = jnp.arange(N, dtype=src.dtype)
    ea = edge_attr[:, 0]
    mean_attr = jnp.mean(ea)
    c1 = jnp.sum(We1[0] * ae1)
    c2 = jnp.sum(We2[0] * ae2)

    pad = EP - ET
    sarr = jnp.concatenate([src, loop, jnp.zeros((pad,), src.dtype)])
    darr = jnp.concatenate([dst, loop, jnp.zeros((pad,), dst.dtype)])
    eab = jnp.concatenate([ea, jnp.full((N,), mean_attr, jnp.float32),
                           jnp.zeros((pad,), jnp.float32)])
    cv1 = jnp.broadcast_to(c1, (L,)).astype(jnp.float32)
    cv2 = jnp.broadcast_to(c2, (L,)).astype(jnp.float32)

    sdarr = (sarr << 16) | darr
    h1, asv1, adv1 = _tc_proj(x, W1, jnp.stack([as1, ad1]))
    maxes1, denomp1, nump1 = _edge_fused1(
        sdarr, eab, cv1, asv1[:, 0], adv1[:, 0], h1)
    h2, asv2, adv2 = _tc_comb1(
        nump1, denomp1.T, maxes1.reshape(NW, L), b1.reshape(1, HID),
        W2, jnp.stack([as2, ad2]).reshape(1, 2))

    maxes2, denomp2, nump2 = _edge_stats2(
        sarr, darr, eab, cv2, asv2[:, 0], adv2[:, 0], h2[:, 0])
    return _tc_comb2(nump2.T, denomp2.T, maxes2.reshape(NW, L),
                     b2.reshape(1, 1))


# speedup vs baseline: 68.4578x; 1.0622x over previous
import functools

import jax
import jax.numpy as jnp
from jax import lax
from jax.experimental import pallas as pl
from jax.experimental.pallas import tpu as pltpu
from jax.experimental.pallas import tpu_sc as plsc

N = 10000
E = 640000
FIN = 1284
HID = 64

NC, NS, L = 2, 16, 16
NW = NC * NS
ET = E + N
CW = 20352
EP = NW * CW
CWC = 2544
NCH = CW // CWC
B = 96
NB = CW // B
NEG = -3.0e38

_SC_PARAMS = pltpu.CompilerParams(needs_layout_passes=False,
                                  use_tc_tiling_on_sc=False)
_MESH = plsc.VectorSubcoreMesh(core_axis_name="c", subcore_axis_name="s",
                               num_cores=NC, num_subcores=NS)

BM = 1000
BMP = 400


def _tc_proj_body(x_ref, w_ref, a_ref, h_ref, asv_ref, adv_ref):
    h = jnp.dot(x_ref[...], w_ref[...], preferred_element_type=jnp.float32)
    h_ref[...] = h
    asv_ref[...] = jnp.sum(h * a_ref[0][None, :], axis=1, keepdims=True)
    adv_ref[...] = jnp.sum(h * a_ref[1][None, :], axis=1, keepdims=True)


def _tc_proj(x, w1, a_sd):
    return pl.pallas_call(
        _tc_proj_body,
        grid=(N // BMP,),
        in_specs=[
            pl.BlockSpec((BMP, FIN), lambda i: (i, 0)),
            pl.BlockSpec((FIN, HID), lambda i: (0, 0)),
            pl.BlockSpec((2, HID), lambda i: (0, 0)),
        ],
        out_specs=[
            pl.BlockSpec((BMP, HID), lambda i: (i, 0)),
            pl.BlockSpec((BMP, 1), lambda i: (i, 0)),
            pl.BlockSpec((BMP, 1), lambda i: (i, 0)),
        ],
        out_shape=[
            jax.ShapeDtypeStruct((N, HID), jnp.float32),
            jax.ShapeDtypeStruct((N, 1), jnp.float32),
            jax.ShapeDtypeStruct((N, 1), jnp.float32),
        ],
    )(x, w1, a_sd)


ER = E // 128
EPR = EP // 128
TR = EPR - ER


def _tc_prep_body(src_ref, dst_ref, ea_ref, we1_ref, ae1_ref, we2_ref,
                  ae2_ref, sd_ref, eab_ref, cpk_ref):
    sd_ref[0:ER, :] = (src_ref[...] << 16) | dst_ref[...]
    eab_ref[0:ER, :] = ea_ref[...]
    mean = jnp.sum(ea_ref[...]) * jnp.float32(1.0 / E)
    nn = (lax.broadcasted_iota(jnp.int32, (TR, 128), 0) * 128
          + lax.broadcasted_iota(jnp.int32, (TR, 128), 1))
    isl = nn < N
    sd_ref[ER:EPR, :] = jnp.where(isl, (nn << 16) | nn, 0)
    eab_ref[ER:EPR, :] = jnp.where(isl, mean, 0.0)
    c1 = jnp.sum(we1_ref[...] * ae1_ref[...])
    c2 = jnp.sum(we2_ref[...] * ae2_ref[...])
    rid = lax.broadcasted_iota(jnp.int32, (8, 128), 0)
    cpk_ref[...] = jnp.where(rid == 0, c1, jnp.where(rid == 1, c2, 0.0))


def _tc_prep(src2, dst2, ea2, we1, ae1, we2, ae2):
    return pl.pallas_call(
        _tc_prep_body,
        out_shape=[
            jax.ShapeDtypeStruct((EPR, 128), jnp.int32),
            jax.ShapeDtypeStruct((EPR, 128), jnp.float32),
            jax.ShapeDtypeStruct((8, 128), jnp.float32),
        ],
    )(src2, dst2, ea2, we1, ae1, we2, ae2)


def _make_edge_stats(with_val):
    out_type = []
    if not with_val:
        out_type.append(jax.ShapeDtypeStruct((EP,), jnp.float32))
    out_type.append(jax.ShapeDtypeStruct((NW * L,), jnp.float32))
    out_type.append(jax.ShapeDtypeStruct((NW, N), jnp.float32))
    if with_val:
        out_type.append(jax.ShapeDtypeStruct((NW, N), jnp.float32))

    scratch = [
        pltpu.VMEM((CW,), jnp.int32),
        pltpu.VMEM((CW,), jnp.float32),
        pltpu.VMEM((N,), jnp.float32),
        pltpu.VMEM((N,), jnp.float32),
        pltpu.VMEM((N,), jnp.float32),
        pltpu.VMEM((CWC,), jnp.float32),
        pltpu.VMEM((L,), jnp.float32),
    ]
    if with_val:
        scratch.append(pltpu.VMEM((N,), jnp.float32))
        scratch.append(pltpu.VMEM((N,), jnp.float32))

    def body(*refs):
        it = iter(refs)
        sdarr, eab, cvec, asv, adv = (next(it) for _ in range(5))
        val = next(it) if with_val else None
        alpha_o = None if with_val else next(it)
        maxes_o, denomp_o = next(it), next(it)
        nump_o = next(it) if with_val else None
        sd_t, alpha_t, asv_t, adv_t, denom_t, eac_b, cv_t = (
            next(it) for _ in range(7))
        val_t = next(it) if with_val else None
        num_t = next(it) if with_val else None

        cid = lax.axis_index("c")
        sid = lax.axis_index("s")
        wid = cid * NS + sid
        base = wid * CW

        pltpu.sync_copy(asv, asv_t)
        pltpu.sync_copy(adv, adv_t)
        if with_val:
            pltpu.sync_copy(val, val_t)
        pltpu.sync_copy(cvec.at[pl.ds(128, L)], cv_t)
        pltpu.sync_copy(sdarr.at[pl.ds(base, CW)], sd_t)
        cvr = cv_t[...]

        def zbody(i, c):
            denom_t[pl.ds(i * L, L)] = jnp.zeros((L,), jnp.float32)
            if with_val:
                num_t[pl.ds(i * L, L)] = jnp.zeros((L,), jnp.float32)
            return c
        lax.fori_loop(0, N // L, zbody, 0)

        def ch_body(c, mx):
            pltpu.sync_copy(eab.at[pl.ds(base + c * CWC, CWC)], eac_b)

            def vbody(j, mx2):
                off = c * CWC + j * L
                pk = sd_t[pl.ds(off, L)]
                sv = lax.shift_right_logical(pk, 16)
                dv = pk & _MASK16
                av = (plsc.load_gather(asv_t, [sv])
                      + plsc.load_gather(adv_t, [dv])
                      + eac_b[pl.ds(j * L, L)] * cvr)
                av = jnp.where(av >= 0.0, av, 0.2 * av)
                gidx = base + off + lax.iota(jnp.int32, 16)
                av = jnp.where(gidx < ET, av, NEG)
                alpha_t[pl.ds(off, L)] = av
                return jnp.maximum(mx2, av)
            return lax.fori_loop(0, CWC // L, vbody, mx)
        mx = lax.fori_loop(0, NCH, ch_body, jnp.full((L,), NEG, jnp.float32))

        eac_b[pl.ds(0, L)] = mx
        pltpu.sync_copy(eac_b.at[pl.ds(0, L)], maxes_o.at[pl.ds(wid * L, L)])

        m = jnp.max(mx)
        msp = jnp.broadcast_to(m, (L,))

        def p2(j, c):
            off = j * L
            av = alpha_t[pl.ds(off, L)]
            ex = jnp.exp(av - msp)
            pk = sd_t[pl.ds(off, L)]
            dv = pk & _MASK16
            plsc.addupdate_scatter(denom_t, [dv], ex)
            if with_val:
                sv = lax.shift_right_logical(pk, 16)
                hs = plsc.load_gather(val_t, [sv])
                plsc.addupdate_scatter(num_t, [dv], ex * hs)
            return c
        lax.fori_loop(0, CW // L, p2, 0)

        if not with_val:
            pltpu.sync_copy(alpha_t, alpha_o.at[pl.ds(base, CW)])
        pltpu.sync_copy(denom_t, denomp_o.at[wid])
        if with_val:
            pltpu.sync_copy(num_t, nump_o.at[wid])

    return functools.partial(
        pl.kernel, out_type=tuple(out_type), mesh=_MESH,
        scratch_types=tuple(scratch), compiler_params=_SC_PARAMS)(body)


_edge_stats2 = _make_edge_stats(with_val=True)


NSL = N // NS
DRC = 25

_FUSED_SCRATCH = (
    pltpu.VMEM((CW,), jnp.int32),
    pltpu.VMEM((2, B), jnp.int32),
    pltpu.VMEM((CW,), jnp.float32),
    pltpu.VMEM((N,), jnp.float32),
    pltpu.VMEM((N,), jnp.float32),
    pltpu.VMEM((N,), jnp.float32),
    pltpu.VMEM((CWC,), jnp.float32),
    pltpu.VMEM((L,), jnp.float32),
    pltpu.VMEM((NS * L,), jnp.float32),
    pltpu.VMEM((B, HID), jnp.float32),
    pltpu.VMEM((B, HID), jnp.float32),
    pltpu.VMEM((DRC, HID), jnp.float32),
    pltpu.VMEM_SHARED((N, HID), jnp.float32),
    pltpu.VMEM_SHARED((NS * L,), jnp.float32),
    pltpu.SemaphoreType.DMA,
    pltpu.SemaphoreType.DMA,
    pltpu.SemaphoreType.DMA,
    pltpu.SemaphoreType.DMA,
)

_MASK16 = 0xFFFF


def _edge_fused1_body(sdarr, eab, cvec, asv, adv, h,
                      maxes_o, denomp_o, nump_o,
                      sd_t, sidx, alpha_t, asv_t, adv_t, denom_t, eac_b,
                      cv_t, mxv_t, rows0, rows1, zrow, acc, mxsh,
                      gs0, gs1, ss0, ss1):
    cid = lax.axis_index("c")
    sid = lax.axis_index("s")
    wid = cid * NS + sid
    base = wid * CW

    pltpu.sync_copy(asv, asv_t)
    pltpu.sync_copy(adv, adv_t)
    pltpu.sync_copy(cvec.at[pl.ds(0, L)], cv_t)
    pltpu.sync_copy(sdarr.at[pl.ds(base, CW)], sd_t)
    cvr = cv_t[...]

    def zb(i, c):
        denom_t[pl.ds(i * L, L)] = jnp.zeros((L,), jnp.float32)
        return c
    lax.fori_loop(0, N // L, zb, 0)

    def ch_body(c, mx):
        pltpu.sync_copy(eab.at[pl.ds(base + c * CWC, CWC)], eac_b)

        def vbody(j, mx2):
            off = c * CWC + j * L
            pk = sd_t[pl.ds(off, L)]
            sv = lax.shift_right_logical(pk, 16)
            dv = pk & _MASK16
            av = (plsc.load_gather(asv_t, [sv])
                  + plsc.load_gather(adv_t, [dv])
                  + eac_b[pl.ds(j * L, L)] * cvr)
            av = jnp.where(av >= 0.0, av, 0.2 * av)
            gidx = base + off + lax.iota(jnp.int32, 16)
            av = jnp.where(gidx < ET, av, NEG)
            alpha_t[pl.ds(off, L)] = av
            return jnp.maximum(mx2, av)
        return lax.fori_loop(0, CWC // L, vbody, mx)
    mx = lax.fori_loop(0, NCH, ch_body, jnp.full((L,), NEG, jnp.float32))

    eac_b[pl.ds(0, L)] = mx
    pltpu.sync_copy(eac_b.at[pl.ds(0, L)], maxes_o.at[pl.ds(wid * L, L)])
    pltpu.sync_copy(eac_b.at[pl.ds(0, L)], mxsh.at[pl.ds(sid * L, L)])

    for r in range(DRC):
        for q in range(HID // L):
            zrow[r, pl.ds(q * L, L)] = jnp.zeros((L,), jnp.float32)

    def zc(k, c):
        pltpu.sync_copy(zrow, acc.at[pl.ds(sid * NSL + k * DRC, DRC)])
        return c
    lax.fori_loop(0, NSL // DRC, zc, 0)
    plsc.subcore_barrier()

    pltpu.sync_copy(mxsh, mxv_t)

    def mb(i, m):
        return jnp.maximum(m, mxv_t[pl.ds(i * L, L)])
    mcv = lax.fori_loop(0, NS, mb, jnp.full((L,), NEG, jnp.float32))
    msp = jnp.broadcast_to(jnp.max(mcv), (L,))

    rows = (rows0, rows1)
    gsems = (gs0, gs1)
    ssems = (ss0, ss1)

    def gather_start(p, blk):
        for q in range(B // L):
            pk = sd_t[pl.ds(blk * B + q * L, L)]
            sidx[p, pl.ds(q * L, L)] = lax.shift_right_logical(pk, 16)
        pltpu.async_copy(h.at[sidx.at[p]], rows[p], gsems[p])

    def gather_wait(p):
        pltpu.make_async_copy(h.at[sidx.at[p]], rows[p], gsems[p]).wait()

    def process(p, blk):
        for j in range(B // L):
            off = blk * B + j * L
            exv = jnp.exp(alpha_t[pl.ds(off, L)] - msp)
            dv = sd_t[pl.ds(off, L)] & _MASK16
            plsc.addupdate_scatter(denom_t, [dv], exv)
            for ee in range(L):
                e = j * L + ee
                esp = jnp.broadcast_to(exv[ee], (L,))
                for q in range(HID // L):
                    rows[p][e, pl.ds(q * L, L)] = (
                        rows[p][e, pl.ds(q * L, L)] * esp)
            pltpu.async_copy(rows[p].at[pl.ds(j * L, L)], acc.at[dv],
                             ssems[p], add=True)

    def scatter_wait(p):
        pltpu.make_async_copy(rows[p], acc.at[sidx.at[p]], ssems[p]).wait()

    gather_start(0, jnp.int32(0))
    gather_start(1, jnp.int32(1))

    def pair_body(gg, carry):
        b0 = gg * 2
        gather_wait(0)
        process(0, b0)
        gather_wait(1)
        process(1, b0 + 1)
        scatter_wait(0)

        @pl.when(b0 + 2 < NB)
        def _():
            gather_start(0, b0 + 2)
        scatter_wait(1)

        @pl.when(b0 + 3 < NB)
        def _():
            gather_start(1, b0 + 3)
        return carry

    lax.fori_loop(0, NB // 2, pair_body, 0)
    plsc.subcore_barrier()

    pltpu.sync_copy(denom_t, denomp_o.at[wid])

    def dr(k, c):
        r0 = sid * NSL + k * DRC
        pltpu.sync_copy(acc.at[pl.ds(r0, DRC)], zrow)
        pltpu.sync_copy(zrow, nump_o.at[cid, pl.ds(r0, DRC)])
        return c
    lax.fori_loop(0, NSL // DRC, dr, 0)


_edge_fused1 = functools.partial(
    pl.kernel,
    out_type=(jax.ShapeDtypeStruct((NW * L,), jnp.float32),
              jax.ShapeDtypeStruct((NW, N), jnp.float32),
              jax.ShapeDtypeStruct((NC, N, HID), jnp.float32)),
    mesh=_MESH, scratch_types=_FUSED_SCRATCH,
    compiler_params=_SC_PARAMS)(_edge_fused1_body)


def _tc_comb1_body(nump_ref, denomp_ref, mx_ref, b1_ref, w2_ref, sc2_ref,
                   h2_ref, asv2_ref, adv2_ref):
    mc0 = jnp.max(mx_ref[0:NS, :])
    mc1 = jnp.max(mx_ref[NS:NW, :])
    g = jnp.maximum(mc0, mc1)
    e0 = jnp.exp(mc0 - g)
    e1 = jnp.exp(mc1 - g)
    dp = denomp_ref[...]
    denom = (e0 * jnp.sum(dp[:, 0:NS], axis=1)
             + e1 * jnp.sum(dp[:, NS:NW], axis=1))
    num = e0 * nump_ref[0] + e1 * nump_ref[1]
    x1 = num / (denom[:, None] + 1e-16) + b1_ref[0][None, :]
    x2 = jnp.where(x1 > 0.0, x1, jnp.exp(x1) - 1.0)
    h2 = jnp.dot(x2, w2_ref[...], preferred_element_type=jnp.float32)
    h2_ref[...] = h2
    asv2_ref[...] = h2 * sc2_ref[:, 0:1]
    adv2_ref[...] = h2 * sc2_ref[:, 1:2]


def _tc_comb1(nump, denomp, mxs, b1, w2, sc2):
    return pl.pallas_call(
        _tc_comb1_body,
        grid=(N // BM,),
        in_specs=[
            pl.BlockSpec((NC, BM, HID), lambda i: (0, i, 0)),
            pl.BlockSpec((BM, NW), lambda i: (i, 0)),
            pl.BlockSpec((NW, L), lambda i: (0, 0)),
            pl.BlockSpec((1, HID), lambda i: (0, 0)),
            pl.BlockSpec((HID, 1), lambda i: (0, 0)),
            pl.BlockSpec((1, 2), lambda i: (0, 0)),
        ],
        out_specs=[
            pl.BlockSpec((BM, 1), lambda i: (i, 0)),
            pl.BlockSpec((BM, 1), lambda i: (i, 0)),
            pl.BlockSpec((BM, 1), lambda i: (i, 0)),
        ],
        out_shape=[
            jax.ShapeDtypeStruct((N, 1), jnp.float32),
            jax.ShapeDtypeStruct((N, 1), jnp.float32),
            jax.ShapeDtypeStruct((N, 1), jnp.float32),
        ],
    )(nump, denomp, mxs, b1, w2, sc2)


def _tc_comb2_body(nump_ref, denomp_ref, mx_ref, b2_ref, out_ref):
    mxw = jnp.max(mx_ref[...], axis=1)
    g = jnp.max(mxw)
    scale = jnp.exp(mxw - g)
    denom = jnp.sum(scale[None, :] * denomp_ref[...], axis=1)
    num = jnp.sum(scale[None, :] * nump_ref[...], axis=1)
    out = num / (denom + 1e-16) + b2_ref[0, 0]
    out = jnp.where(out > 0.0, out, 0.01 * out)
    out_ref[...] = out[:, None]


def _tc_comb2(nump, denomp, mxs, b2):
    return pl.pallas_call(
        _tc_comb2_body,
        grid=(N // BM,),
        in_specs=[
            pl.BlockSpec((BM, NW), lambda i: (i, 0)),
            pl.BlockSpec((BM, NW), lambda i: (i, 0)),
            pl.BlockSpec((NW, L), lambda i: (0, 0)),
            pl.BlockSpec((1, 1), lambda i: (0, 0)),
        ],
        out_specs=pl.BlockSpec((BM, 1), lambda i: (i, 0)),
        out_shape=jax.ShapeDtypeStruct((N, 1), jnp.float32),
    )(nump, denomp, mxs, b2)


def kernel(x, edge_index, edge_attr, W1, as1, ad1, We1, ae1, b1,
           W2, as2, ad2, We2, ae2, b2):
    src2 = edge_index[0].reshape(ER, 128)
    dst2 = edge_index[1].reshape(ER, 128)
    ea2 = edge_attr.reshape(ER, 128)

    sd2, eab2, cpk = _tc_prep(
        src2, dst2, ea2, We1, ae1.reshape(1, HID),
        We2, ae2.reshape(1, 1))
    sdarr = sd2.reshape(EP)
    eab = eab2.reshape(EP)
    cpack = cpk.reshape(1024)

    h1, asv1, adv1 = _tc_proj(x, W1, jnp.stack([as1, ad1]))
    maxes1, denomp1, nump1 = _edge_fused1(
        sdarr, eab, cpack, asv1[:, 0], adv1[:, 0], h1)
    h2, asv2, adv2 = _tc_comb1(
        nump1, denomp1.T, maxes1.reshape(NW, L), b1.reshape(1, HID),
        W2, jnp.stack([as2, ad2]).reshape(1, 2))

    maxes2, denomp2, nump2 = _edge_stats2(
        sdarr, eab, cpack, asv2[:, 0], adv2[:, 0], h2[:, 0])
    return _tc_comb2(nump2.T, denomp2.T, maxes2.reshape(NW, L),
                     b2.reshape(1, 1))
